# Initial kernel scaffold; baseline (speedup 1.0000x reference)
import jax
import jax.numpy as jnp
from jax.experimental import pallas as pl


def kernel(context, adj, W0, as0, ad0, We0, ae0, b0, W1, as1, ad1, We1, ae1, b1, W2, as2, ad2, We2, ae2, b2):
    raise NotImplementedError("write your pallas kernel here")


# speedup vs baseline: 347.0523x; 347.0523x over previous
import jax
import jax.numpy as jnp
from jax.experimental import pallas as pl

_B, _N, _F_IN, _HID, _HEADS = 16, 64, 256, 256, 8
_C1 = _HID // _HEADS
_C2 = _N
_NEG = -1e30


def _gat_layer(x, esT, maskT, W_ref, As_ref, Ad_ref, we_ref, b_ref, heads, ch,
               concat):
    xs = jnp.dot(x, W_ref[...], preferred_element_type=jnp.float32)
    asrc = jnp.dot(xs, As_ref[...], preferred_element_type=jnp.float32)
    adst = jnp.dot(xs, Ad_ref[...], preferred_element_type=jnp.float32)
    asrcT = asrc.T
    outs = []
    acc = None
    for h in range(heads):
        row_src = asrcT[h:h + 1, :]
        col_dst = adst[:, h:h + 1]
        weh = we_ref[0:1, h:h + 1]
        lg = row_src + col_dst + esT * weh
        lg = jnp.where(lg >= 0, lg, 0.2 * lg)
        lg = jnp.where(maskT, lg, _NEG)
        m = jnp.max(lg, axis=1, keepdims=True)
        p = jnp.where(maskT, jnp.exp(lg - m), 0.0)
        s = jnp.sum(p, axis=1, keepdims=True)
        P = p / (s + 1e-16)
        out_h = jnp.dot(P, xs[:, h * ch:(h + 1) * ch],
                        preferred_element_type=jnp.float32)
        if concat:
            outs.append(out_h)
        else:
            acc = out_h if acc is None else acc + out_h
    out = jnp.concatenate(outs, axis=1) if concat else acc * (1.0 / heads)
    return out + b_ref[...]


def _gnn_kernel(x_ref, adjT_ref,
                W0_ref, As0_ref, Ad0_ref, we0_ref, b0_ref,
                W1_ref, As1_ref, Ad1_ref, we1_ref, b1_ref,
                W2_ref, As2_ref, Ad2_ref, we2_ref, b2_ref,
                out_ref):
    x = x_ref[...]
    adjT = adjT_ref[0]

    ii = jax.lax.broadcasted_iota(jnp.int32, (_N, _N), 1)
    jj = jax.lax.broadcasted_iota(jnp.int32, (_N, _N), 0)
    diag = ii == jj
    offmask = jnp.logical_and(jnp.logical_not(diag), adjT != 0.0)
    cnt = jnp.sum(offmask.astype(jnp.float32), axis=1, keepdims=True)
    ssum = jnp.sum(jnp.where(offmask, adjT, 0.0), axis=1, keepdims=True)
    loop_attr = jnp.where(cnt > 0.0, ssum / jnp.maximum(cnt, 1.0), 0.0)
    esT = jnp.where(diag, loop_attr, adjT)
    maskT = jnp.logical_or(diag, offmask)

    h = _gat_layer(x, esT, maskT, W0_ref, As0_ref, Ad0_ref, we0_ref, b0_ref,
                   _HEADS, _C1, True)
    h = jnp.maximum(h, 0.0)
    h = _gat_layer(h, esT, maskT, W1_ref, As1_ref, Ad1_ref, we1_ref, b1_ref,
                   _HEADS, _C1, True)
    h = jnp.maximum(h, 0.0)
    h = _gat_layer(h, esT, maskT, W2_ref, As2_ref, Ad2_ref, we2_ref, b2_ref,
                   _HEADS, _C2, False)
    out_ref[0] = jax.nn.sigmoid(h)


def _head_selector(a, heads, ch):
    eye = jnp.eye(heads, dtype=a.dtype)
    return (a[:, :, None] * eye[:, None, :]).reshape(heads * ch, heads)


def kernel(context, adj, W0, as0, ad0, We0, ae0, b0,
           W1, as1, ad1, We1, ae1, b1,
           W2, as2, ad2, We2, ae2, b2):
    x = context.reshape(_B * _N, _F_IN)
    adjT = adj.transpose(0, 2, 1)

    As0 = _head_selector(as0, _HEADS, _C1)
    Ad0 = _head_selector(ad0, _HEADS, _C1)
    As1 = _head_selector(as1, _HEADS, _C1)
    Ad1 = _head_selector(ad1, _HEADS, _C1)
    As2 = _head_selector(as2, _HEADS, _C2)
    Ad2 = _head_selector(ad2, _HEADS, _C2)
    we0f = (We0.reshape(_HEADS, _C1) * ae0).sum(-1).reshape(1, _HEADS)
    we1f = (We1.reshape(_HEADS, _C1) * ae1).sum(-1).reshape(1, _HEADS)
    we2f = (We2.reshape(_HEADS, _C2) * ae2).sum(-1).reshape(1, _HEADS)
    b0r = b0.reshape(1, _HID)
    b1r = b1.reshape(1, _HID)
    b2r = b2.reshape(1, _C2)

    def fixed(shape):
        return pl.BlockSpec(shape, lambda b: tuple(0 for _ in shape))

    att = pl.pallas_call(
        _gnn_kernel,
        grid=(_B,),
        in_specs=[
            pl.BlockSpec((_N, _F_IN), lambda b: (b, 0)),
            pl.BlockSpec((1, _N, _N), lambda b: (b, 0, 0)),
            fixed((_F_IN, _HID)), fixed((_HID, _HEADS)), fixed((_HID, _HEADS)),
            fixed((1, _HEADS)), fixed((1, _HID)),
            fixed((_HID, _HID)), fixed((_HID, _HEADS)), fixed((_HID, _HEADS)),
            fixed((1, _HEADS)), fixed((1, _HID)),
            fixed((_HID, _HEADS * _C2)), fixed((_HEADS * _C2, _HEADS)),
            fixed((_HEADS * _C2, _HEADS)), fixed((1, _HEADS)), fixed((1, _C2)),
        ],
        out_specs=pl.BlockSpec((1, _N, _N), lambda b: (b, 0, 0)),
        out_shape=jax.ShapeDtypeStruct((_B, _N, _N), jnp.float32),
    )(x, adjT, W0, As0, Ad0, we0f, b0r, W1, As1, Ad1, we1f, b1r,
      W2, As2, Ad2, we2f, b2r)
    return att


# speedup vs baseline: 361.0420x; 1.0403x over previous
import jax
import jax.numpy as jnp
from jax.experimental import pallas as pl

_B, _N, _F_IN, _HID, _HEADS = 16, 64, 256, 256, 8
_C1 = _HID // _HEADS
_C2 = _N
_NEG = -1e30
_G = 16
_BPG = _B // _G


def _gat_layer(x, esT_l, maskadd_l, W_ref, As_ref, Ad_ref, we_ref, b_ref,
               heads, ch, concat):
    xs = jnp.dot(x, W_ref[...], preferred_element_type=jnp.float32)
    asrc = jnp.dot(xs, As_ref[...], preferred_element_type=jnp.float32)
    adst = jnp.dot(xs, Ad_ref[...], preferred_element_type=jnp.float32)
    asrcT = asrc.T
    b_outs = []
    for b in range(_BPG):
        r0 = b * _N
        outs = []
        acc = None
        for h in range(heads):
            row_src = asrcT[h:h + 1, r0:r0 + _N]
            col_dst = adst[r0:r0 + _N, h:h + 1]
            weh = we_ref[h:h + 1, 0:1]
            lg = esT_l[b] * weh + row_src + col_dst
            lg = jnp.maximum(lg, 0.2 * lg) + maskadd_l[b]
            m = jnp.max(lg, axis=1, keepdims=True)
            p = jnp.exp(lg - m)
            s = jnp.sum(p, axis=1, keepdims=True)
            agg = jnp.dot(p, xs[r0:r0 + _N, h * ch:(h + 1) * ch],
                          preferred_element_type=jnp.float32)
            out_h = agg / s
            if concat:
                outs.append(out_h)
            else:
                acc = out_h if acc is None else acc + out_h
        out_b = jnp.concatenate(outs, axis=1) if concat else acc * (1.0 / heads)
        b_outs.append(out_b)
    out = jnp.concatenate(b_outs, axis=0) if _BPG > 1 else b_outs[0]
    return out + b_ref[...]


def _gnn_kernel(x_ref, adjT_ref,
                W0_ref, As0_ref, Ad0_ref, we0_ref, b0_ref,
                W1_ref, As1_ref, Ad1_ref, we1_ref, b1_ref,
                W2_ref, As2_ref, Ad2_ref, we2_ref, b2_ref,
                out_ref):
    x = x_ref[...]

    ii = jax.lax.broadcasted_iota(jnp.int32, (_N, _N), 1)
    jj = jax.lax.broadcasted_iota(jnp.int32, (_N, _N), 0)
    diag = ii == jj
    esT_l, maskadd_l = [], []
    for b in range(_BPG):
        adjT = adjT_ref[b]
        offmask = jnp.logical_and(jnp.logical_not(diag), adjT != 0.0)
        cnt = jnp.sum(offmask.astype(jnp.float32), axis=1, keepdims=True)
        ssum = jnp.sum(jnp.where(offmask, adjT, 0.0), axis=1, keepdims=True)
        loop_attr = jnp.where(cnt > 0.0, ssum / jnp.maximum(cnt, 1.0), 0.0)
        esT_l.append(jnp.where(diag, loop_attr, adjT))
        maskadd_l.append(
            jnp.where(jnp.logical_or(diag, offmask), 0.0, _NEG))

    h = _gat_layer(x, esT_l, maskadd_l, W0_ref, As0_ref, Ad0_ref, we0_ref,
                   b0_ref, _HEADS, _C1, True)
    h = jnp.maximum(h, 0.0)
    h = _gat_layer(h, esT_l, maskadd_l, W1_ref, As1_ref, Ad1_ref, we1_ref,
                   b1_ref, _HEADS, _C1, True)
    h = jnp.maximum(h, 0.0)
    h = _gat_layer(h, esT_l, maskadd_l, W2_ref, As2_ref, Ad2_ref, we2_ref,
                   b2_ref, _HEADS, _C2, False)
    out = jax.nn.sigmoid(h)
    for b in range(_BPG):
        out_ref[b] = out[b * _N:(b + 1) * _N, :]


def _head_selector(a, heads, ch):
    eye = jnp.eye(heads, dtype=a.dtype)
    return (a[:, :, None] * eye[:, None, :]).reshape(heads * ch, heads)


def kernel(context, adj, W0, as0, ad0, We0, ae0, b0,
           W1, as1, ad1, We1, ae1, b1,
           W2, as2, ad2, We2, ae2, b2):
    x = context.reshape(_B * _N, _F_IN)
    adjT = adj.transpose(0, 2, 1)

    As0 = _head_selector(as0, _HEADS, _C1)
    Ad0 = _head_selector(ad0, _HEADS, _C1)
    As1 = _head_selector(as1, _HEADS, _C1)
    Ad1 = _head_selector(ad1, _HEADS, _C1)
    As2 = _head_selector(as2, _HEADS, _C2)
    Ad2 = _head_selector(ad2, _HEADS, _C2)
    we0f = (We0.reshape(_HEADS, _C1) * ae0).sum(-1).reshape(_HEADS, 1)
    we1f = (We1.reshape(_HEADS, _C1) * ae1).sum(-1).reshape(_HEADS, 1)
    we2f = (We2.reshape(_HEADS, _C2) * ae2).sum(-1).reshape(_HEADS, 1)
    b0r = b0.reshape(1, _HID)
    b1r = b1.reshape(1, _HID)
    b2r = b2.reshape(1, _C2)

    def fixed(shape):
        return pl.BlockSpec(shape, lambda g: tuple(0 for _ in shape))

    att = pl.pallas_call(
        _gnn_kernel,
        grid=(_G,),
        in_specs=[
            pl.BlockSpec((_BPG * _N, _F_IN), lambda g: (g, 0)),
            pl.BlockSpec((_BPG, _N, _N), lambda g: (g, 0, 0)),
            fixed((_F_IN, _HID)), fixed((_HID, _HEADS)), fixed((_HID, _HEADS)),
            fixed((_HEADS, 1)), fixed((1, _HID)),
            fixed((_HID, _HID)), fixed((_HID, _HEADS)), fixed((_HID, _HEADS)),
            fixed((_HEADS, 1)), fixed((1, _HID)),
            fixed((_HID, _HEADS * _C2)), fixed((_HEADS * _C2, _HEADS)),
            fixed((_HEADS * _C2, _HEADS)), fixed((_HEADS, 1)), fixed((1, _C2)),
        ],
        out_specs=pl.BlockSpec((_BPG, _N, _N), lambda g: (g, 0, 0)),
        out_shape=jax.ShapeDtypeStruct((_B, _N, _N), jnp.float32),
    )(x, adjT, W0, As0, Ad0, we0f, b0r, W1, As1, Ad1, we1f, b1r,
      W2, As2, Ad2, we2f, b2r)
    return att
